# baseline (speedup 1.0000x reference)
import jax
import jax.numpy as jnp
from jax import lax
from jax.experimental import pallas as pl
from jax.experimental.pallas import tpu as pltpu
from jax.experimental.pallas import tpu_sc as plsc

_NUM_CLASSES = 1000
_ALPHA = 0.3

_N = 16384
_D = 128
_NC = 2
_NS = 16
_NW = _NC * _NS
_RPT = _N // _NW
_CHUNK = 128
_NCHUNK = _RPT // _CHUNK
_CPAD = 1024
_ZROWS = _CPAD // _NS
_CW = 16


def _vmesh():
  return plsc.VectorSubcoreMesh(core_axis_name="c", subcore_axis_name="s")


def _sc_stats_body(x_hbm, xsq_hbm, y2_hbm, zeros_hbm, zeros16_hbm, ones_hbm,
                   s_out, s2_out, cnt_out,
                   s_sh, s2_sh, cnt_sh, xa, xb, qa, qb, idx_v, ones_v, sem):
  core = lax.axis_index("c")
  sid = lax.axis_index("s")
  wid = sid * _NC + core
  base = wid * _RPT
  zsl = pl.ds(sid * _ZROWS, _ZROWS)

  pltpu.sync_copy(zeros_hbm, s_sh.at[zsl])
  pltpu.sync_copy(zeros_hbm, s2_sh.at[zsl])
  pltpu.sync_copy(zeros16_hbm, cnt_sh.at[zsl])
  pltpu.sync_copy(ones_hbm, ones_v)
  pltpu.sync_copy(y2_hbm.at[pl.ds(wid * _NCHUNK, _NCHUNK)], idx_v)
  plsc.subcore_barrier()

  bufs = ((xa, qa), (xb, qb))
  cp_x = pltpu.async_copy(x_hbm.at[pl.ds(base, _CHUNK)], xa, sem)
  cp_q = pltpu.async_copy(xsq_hbm.at[pl.ds(base, _CHUNK)], qa, sem)
  pending = (cp_x, cp_q)
  for j in range(_NCHUNK):
    cur_x, cur_q = bufs[j % 2]
    pending[0].wait()
    pending[1].wait()
    if j + 1 < _NCHUNK:
      nxt_x, nxt_q = bufs[(j + 1) % 2]
      off = base + (j + 1) * _CHUNK
      pending = (pltpu.async_copy(x_hbm.at[pl.ds(off, _CHUNK)], nxt_x, sem),
                 pltpu.async_copy(xsq_hbm.at[pl.ds(off, _CHUNK)], nxt_q, sem))
    pltpu.sync_copy(cur_x, s_sh.at[idx_v.at[j]], add=True)
    pltpu.sync_copy(cur_q, s2_sh.at[idx_v.at[j]], add=True)
    pltpu.sync_copy(ones_v, cnt_sh.at[idx_v.at[j]], add=True)

  plsc.subcore_barrier()
  pltpu.sync_copy(s_sh.at[zsl], s_out.at[core, zsl])
  pltpu.sync_copy(s2_sh.at[zsl], s2_out.at[core, zsl])
  pltpu.sync_copy(cnt_sh.at[zsl], cnt_out.at[core, zsl])


def _sc_gather_body(tab_hbm, ny2_hbm, g_out, g_v, idx_v, sg, ss):
  core = lax.axis_index("c")
  sid = lax.axis_index("s")
  wid = sid * _NC + core
  base = wid * _NCHUNK

  pltpu.sync_copy(ny2_hbm.at[pl.ds(base, _NCHUNK)], idx_v)
  gath = [None, None]
  stor = [None, None]
  for j in range(2):
    gath[j] = pltpu.async_copy(tab_hbm.at[idx_v.at[j]], g_v.at[j], sg)
  for j in range(_NCHUNK):
    b = j % 2
    gath[b].wait()
    stor[b] = pltpu.async_copy(g_v.at[b], g_out.at[base + j], ss)
    if j + 2 < _NCHUNK:
      stor[b].wait()
      gath[b] = pltpu.async_copy(tab_hbm.at[idx_v.at[j + 2]], g_v.at[b], sg)
  stor[0].wait()
  stor[1].wait()


def _tc_square_body(x_ref, o_ref):
  x = x_ref[...]
  o_ref[...] = x * x


def _tc_finalize_body(s_ref, s2_ref, c_ref, o_ref):
  s = s_ref[0] + s_ref[1]
  s2 = s2_ref[0] + s2_ref[1]
  cnt = c_ref[0, :, 0:1] + c_ref[1, :, 0:1]
  mean = s / cnt
  var = (s2 - cnt * mean * mean) / (cnt - 1.0)
  std = jnp.sqrt(jnp.maximum(var, 0.0))
  o_ref[:, 0:_D] = mean
  o_ref[:, _D:2 * _D] = std


def _tc_combine_body(x_ref, e_ref, g_ref, o_ref):
  g = g_ref[...]
  noise = g[:, 0:_D] + g[:, _D:2 * _D] * e_ref[...]
  o_ref[...] = (1.0 - _ALPHA) * x_ref[...] + _ALPHA * noise


def _segment_stats(x, xsq, y2, zeros, zeros16, ones):
  sds = jax.ShapeDtypeStruct((_NC, _CPAD, _D), jnp.float32)
  cds = jax.ShapeDtypeStruct((_NC, _CPAD, _CW), jnp.float32)
  k = pl.kernel(
      _sc_stats_body,
      out_type=(sds, sds, cds),
      mesh=_vmesh(),
      scratch_types=[
          pltpu.VMEM_SHARED((_CPAD, _D), jnp.float32),
          pltpu.VMEM_SHARED((_CPAD, _D), jnp.float32),
          pltpu.VMEM_SHARED((_CPAD, _CW), jnp.float32),
          pltpu.VMEM((_CHUNK, _D), jnp.float32),
          pltpu.VMEM((_CHUNK, _D), jnp.float32),
          pltpu.VMEM((_CHUNK, _D), jnp.float32),
          pltpu.VMEM((_CHUNK, _D), jnp.float32),
          pltpu.VMEM((_NCHUNK, _CHUNK), jnp.int32),
          pltpu.VMEM((_CHUNK, _CW), jnp.float32),
          pltpu.SemaphoreType.DMA,
      ],
  )
  return k(x, xsq, y2, zeros, zeros16, ones)


def _gather_rows(tab, ny2):
  k = pl.kernel(
      _sc_gather_body,
      out_type=jax.ShapeDtypeStruct((_N // _CHUNK, _CHUNK, 2 * _D),
                                    jnp.float32),
      mesh=_vmesh(),
      scratch_types=[
          pltpu.VMEM((2, _CHUNK, 2 * _D), jnp.float32),
          pltpu.VMEM((_NCHUNK, _CHUNK), jnp.int32),
          pltpu.SemaphoreType.DMA,
          pltpu.SemaphoreType.DMA,
      ],
  )
  return k(tab, ny2)


def _resample(y, key):
  k1, k2 = jax.random.split(key)
  perm = jax.random.permutation(k1, y.shape[0])
  ny = y[perm]

  k = k2
  for _ in range(4):
    k, sub = jax.random.split(k)
    rand = jax.random.randint(sub, y.shape, 0, _NUM_CLASSES).astype(y.dtype)
    ny = jnp.where(ny == y, rand, ny)

  def cond(state):
    n, _ = state
    return jnp.any(n == y)

  def body(state):
    n, kk = state
    kk, sub = jax.random.split(kk)
    rand = jax.random.randint(sub, y.shape, 0, _NUM_CLASSES).astype(y.dtype)
    n = jnp.where(n == y, rand, n)
    return (n, kk)

  ny, _ = jax.lax.while_loop(cond, body, (ny, k))
  return ny


def kernel(x, y):
  k_perm, k_noise = jax.random.split(jax.random.key(42))
  new_y = _resample(y, k_perm)
  eps = jax.random.normal(k_noise, x.shape, dtype=x.dtype)

  blk = 1024
  xsq = pl.pallas_call(
      _tc_square_body,
      grid=(_N // blk,),
      in_specs=[pl.BlockSpec((blk, _D), lambda i: (i, 0))],
      out_specs=pl.BlockSpec((blk, _D), lambda i: (i, 0)),
      out_shape=jax.ShapeDtypeStruct((_N, _D), jnp.float32),
  )(x)

  zeros = jnp.zeros((_ZROWS, _D), jnp.float32)
  zeros16 = jnp.zeros((_ZROWS, _CW), jnp.float32)
  ones = jnp.ones((_CHUNK, _CW), jnp.float32)
  y2 = y.reshape(_N // _CHUNK, _CHUNK)
  s_p, s2_p, cnt_p = _segment_stats(x, xsq, y2, zeros, zeros16, ones)

  tab = pl.pallas_call(
      _tc_finalize_body,
      out_shape=jax.ShapeDtypeStruct((_CPAD, 2 * _D), jnp.float32),
  )(s_p, s2_p, cnt_p)

  ny2 = new_y.reshape(_N // _CHUNK, _CHUNK)
  gmgs = _gather_rows(tab, ny2).reshape(_N, 2 * _D)

  out = pl.pallas_call(
      _tc_combine_body,
      grid=(_N // blk,),
      in_specs=[
          pl.BlockSpec((blk, _D), lambda i: (i, 0)),
          pl.BlockSpec((blk, _D), lambda i: (i, 0)),
          pl.BlockSpec((blk, 2 * _D), lambda i: (i, 0)),
      ],
      out_specs=pl.BlockSpec((blk, _D), lambda i: (i, 0)),
      out_shape=jax.ShapeDtypeStruct((_N, _D), jnp.float32),
  )(x, eps, gmgs)

  return (out, new_y)


# speedup vs baseline: 1.2403x; 1.2403x over previous
import jax
import jax.numpy as jnp
from jax import lax
from jax.experimental import pallas as pl
from jax.experimental.pallas import tpu as pltpu
from jax.experimental.pallas import tpu_sc as plsc

_NUM_CLASSES = 1000
_ALPHA = 0.3

_N = 16384
_D = 128
_NC = 2
_NS = 16
_NW = _NC * _NS
_RPT = _N // _NW
_CHUNK = 128
_NCHUNK = _RPT // _CHUNK
_CPAD = 1024
_ZROWS = _CPAD // _NS
_CW = 16


def _vmesh():
  return plsc.VectorSubcoreMesh(core_axis_name="c", subcore_axis_name="s")


def _sc_stats_body(x_hbm, xsq_hbm, y2_hbm, zeros_hbm, zeros16_hbm, ones_hbm,
                   s_out, s2_out, cnt_out,
                   s_sh, s2_sh, cnt_sh, xa, xb, qa, qb, idx_v, ones_v, sem):
  core = lax.axis_index("c")
  sid = lax.axis_index("s")
  wid = sid * _NC + core
  base = wid * _RPT
  zsl = pl.ds(sid * _ZROWS, _ZROWS)

  pltpu.sync_copy(zeros_hbm, s_sh.at[zsl])
  pltpu.sync_copy(zeros_hbm, s2_sh.at[zsl])
  pltpu.sync_copy(zeros16_hbm, cnt_sh.at[zsl])
  pltpu.sync_copy(ones_hbm, ones_v)
  pltpu.sync_copy(y2_hbm.at[pl.ds(wid * _NCHUNK, _NCHUNK)], idx_v)
  plsc.subcore_barrier()

  bufs = ((xa, qa), (xb, qb))
  cp_x = pltpu.async_copy(x_hbm.at[pl.ds(base, _CHUNK)], xa, sem)
  cp_q = pltpu.async_copy(xsq_hbm.at[pl.ds(base, _CHUNK)], qa, sem)
  pending = (cp_x, cp_q)
  for j in range(_NCHUNK):
    cur_x, cur_q = bufs[j % 2]
    pending[0].wait()
    pending[1].wait()
    if j + 1 < _NCHUNK:
      nxt_x, nxt_q = bufs[(j + 1) % 2]
      off = base + (j + 1) * _CHUNK
      pending = (pltpu.async_copy(x_hbm.at[pl.ds(off, _CHUNK)], nxt_x, sem),
                 pltpu.async_copy(xsq_hbm.at[pl.ds(off, _CHUNK)], nxt_q, sem))
    pltpu.sync_copy(cur_x, s_sh.at[idx_v.at[j]], add=True)
    pltpu.sync_copy(cur_q, s2_sh.at[idx_v.at[j]], add=True)
    pltpu.sync_copy(ones_v, cnt_sh.at[idx_v.at[j]], add=True)

  plsc.subcore_barrier()
  pltpu.sync_copy(s_sh.at[zsl], s_out.at[core, zsl])
  pltpu.sync_copy(s2_sh.at[zsl], s2_out.at[core, zsl])
  pltpu.sync_copy(cnt_sh.at[zsl], cnt_out.at[core, zsl])


def _sc_gather_body(tab_hbm, ny2_hbm, g_out, g_v, idx_v, sg, ss):
  core = lax.axis_index("c")
  sid = lax.axis_index("s")
  wid = sid * _NC + core
  base = wid * _NCHUNK

  pltpu.sync_copy(ny2_hbm.at[pl.ds(base, _NCHUNK)], idx_v)
  gath = [None, None]
  stor = [None, None]
  for j in range(2):
    gath[j] = pltpu.async_copy(tab_hbm.at[idx_v.at[j]], g_v.at[j], sg)
  for j in range(_NCHUNK):
    b = j % 2
    gath[b].wait()
    stor[b] = pltpu.async_copy(g_v.at[b], g_out.at[base + j], ss)
    if j + 2 < _NCHUNK:
      stor[b].wait()
      gath[b] = pltpu.async_copy(tab_hbm.at[idx_v.at[j + 2]], g_v.at[b], sg)
  stor[0].wait()
  stor[1].wait()


def _tc_square_body(x_ref, o_ref):
  x = x_ref[...]
  o_ref[...] = x * x


def _tc_finalize_body(s_ref, s2_ref, c_ref, o_ref):
  s = s_ref[0] + s_ref[1]
  s2 = s2_ref[0] + s2_ref[1]
  cnt = c_ref[0, :, 0:1] + c_ref[1, :, 0:1]
  mean = s / cnt
  var = (s2 - cnt * mean * mean) / (cnt - 1.0)
  std = jnp.sqrt(jnp.maximum(var, 0.0))
  o_ref[:, 0:_D] = mean
  o_ref[:, _D:2 * _D] = std


def _tc_combine_body(x_ref, e_ref, g_ref, o_ref):
  g = g_ref[...]
  noise = g[:, 0:_D] + g[:, _D:2 * _D] * e_ref[...]
  o_ref[...] = (1.0 - _ALPHA) * x_ref[...] + _ALPHA * noise


def _segment_stats(x, xsq, y2, zeros, zeros16, ones):
  sds = jax.ShapeDtypeStruct((_NC, _CPAD, _D), jnp.float32)
  cds = jax.ShapeDtypeStruct((_NC, _CPAD, _CW), jnp.float32)
  k = pl.kernel(
      _sc_stats_body,
      out_type=(sds, sds, cds),
      mesh=_vmesh(),
      scratch_types=[
          pltpu.VMEM_SHARED((_CPAD, _D), jnp.float32),
          pltpu.VMEM_SHARED((_CPAD, _D), jnp.float32),
          pltpu.VMEM_SHARED((_CPAD, _CW), jnp.float32),
          pltpu.VMEM((_CHUNK, _D), jnp.float32),
          pltpu.VMEM((_CHUNK, _D), jnp.float32),
          pltpu.VMEM((_CHUNK, _D), jnp.float32),
          pltpu.VMEM((_CHUNK, _D), jnp.float32),
          pltpu.VMEM((_NCHUNK, _CHUNK), jnp.int32),
          pltpu.VMEM((_CHUNK, _CW), jnp.float32),
          pltpu.SemaphoreType.DMA,
      ],
  )
  return k(x, xsq, y2, zeros, zeros16, ones)


def _gather_rows(tab, ny2):
  k = pl.kernel(
      _sc_gather_body,
      out_type=jax.ShapeDtypeStruct((_N // _CHUNK, _CHUNK, 2 * _D),
                                    jnp.float32),
      mesh=_vmesh(),
      scratch_types=[
          pltpu.VMEM((2, _CHUNK, 2 * _D), jnp.float32),
          pltpu.VMEM((_NCHUNK, _CHUNK), jnp.int32),
          pltpu.SemaphoreType.DMA,
          pltpu.SemaphoreType.DMA,
      ],
  )
  return k(tab, ny2)


def _resample(y, key):
  k1, k2 = jax.random.split(key)
  perm = jax.random.permutation(k1, y.shape[0])
  ny = y[perm]

  k = k2
  for _ in range(1):
    k, sub = jax.random.split(k)
    rand = jax.random.randint(sub, y.shape, 0, _NUM_CLASSES).astype(y.dtype)
    ny = jnp.where(ny == y, rand, ny)

  def cond(state):
    n, _ = state
    return jnp.any(n == y)

  def body(state):
    n, kk = state
    kk, sub = jax.random.split(kk)
    rand = jax.random.randint(sub, y.shape, 0, _NUM_CLASSES).astype(y.dtype)
    n = jnp.where(n == y, rand, n)
    return (n, kk)

  ny, _ = jax.lax.while_loop(cond, body, (ny, k))
  return ny


def kernel(x, y):
  k_perm, k_noise = jax.random.split(jax.random.key(42))
  new_y = _resample(y, k_perm)
  eps = jax.random.normal(k_noise, x.shape, dtype=x.dtype)

  blk = 1024
  xsq = pl.pallas_call(
      _tc_square_body,
      grid=(_N // blk,),
      in_specs=[pl.BlockSpec((blk, _D), lambda i: (i, 0))],
      out_specs=pl.BlockSpec((blk, _D), lambda i: (i, 0)),
      out_shape=jax.ShapeDtypeStruct((_N, _D), jnp.float32),
  )(x)

  zeros = jnp.zeros((_ZROWS, _D), jnp.float32)
  zeros16 = jnp.zeros((_ZROWS, _CW), jnp.float32)
  ones = jnp.ones((_CHUNK, _CW), jnp.float32)
  y2 = y.reshape(_N // _CHUNK, _CHUNK)
  s_p, s2_p, cnt_p = _segment_stats(x, xsq, y2, zeros, zeros16, ones)

  tab = pl.pallas_call(
      _tc_finalize_body,
      out_shape=jax.ShapeDtypeStruct((_CPAD, 2 * _D), jnp.float32),
  )(s_p, s2_p, cnt_p)

  ny2 = new_y.reshape(_N // _CHUNK, _CHUNK)
  gmgs = _gather_rows(tab, ny2).reshape(_N, 2 * _D)

  out = pl.pallas_call(
      _tc_combine_body,
      grid=(_N // blk,),
      in_specs=[
          pl.BlockSpec((blk, _D), lambda i: (i, 0)),
          pl.BlockSpec((blk, _D), lambda i: (i, 0)),
          pl.BlockSpec((blk, 2 * _D), lambda i: (i, 0)),
      ],
      out_specs=pl.BlockSpec((blk, _D), lambda i: (i, 0)),
      out_shape=jax.ShapeDtypeStruct((_N, _D), jnp.float32),
  )(x, eps, gmgs)

  return (out, new_y)


# speedup vs baseline: 1.2534x; 1.0106x over previous
import jax
import jax.numpy as jnp
from jax import lax
from jax.experimental import pallas as pl
from jax.experimental.pallas import tpu as pltpu
from jax.experimental.pallas import tpu_sc as plsc

_NUM_CLASSES = 1000
_ALPHA = 0.3

_N = 16384
_D = 128
_NC = 2
_NS = 16
_NW = _NC * _NS
_RPT = _N // _NW
_CHUNK = 128
_NCHUNK = _RPT // _CHUNK
_CPAD = 1024
_ZROWS = _CPAD // _NS
_CW = 16


def _vmesh():
  return plsc.VectorSubcoreMesh(core_axis_name="c", subcore_axis_name="s")


def _sc_stats_body(x_hbm, xsq_hbm, y2_hbm, zeros_hbm, zeros16_hbm, ones_hbm,
                   s_out, s2_out, cnt_out,
                   s_sh, s2_sh, cnt_sh, xa, xb, qa, qb, idx_v, ones_v, sem):
  core = lax.axis_index("c")
  sid = lax.axis_index("s")
  wid = sid * _NC + core
  base = wid * _RPT
  zsl = pl.ds(sid * _ZROWS, _ZROWS)

  pltpu.sync_copy(zeros_hbm, s_sh.at[zsl])
  pltpu.sync_copy(zeros_hbm, s2_sh.at[zsl])
  pltpu.sync_copy(zeros16_hbm, cnt_sh.at[zsl])
  pltpu.sync_copy(ones_hbm, ones_v)
  pltpu.sync_copy(y2_hbm.at[pl.ds(wid * _NCHUNK, _NCHUNK)], idx_v)
  plsc.subcore_barrier()

  bufs = ((xa, qa), (xb, qb))
  cp_x = pltpu.async_copy(x_hbm.at[pl.ds(base, _CHUNK)], xa, sem)
  cp_q = pltpu.async_copy(xsq_hbm.at[pl.ds(base, _CHUNK)], qa, sem)
  pending = (cp_x, cp_q)
  for j in range(_NCHUNK):
    cur_x, cur_q = bufs[j % 2]
    pending[0].wait()
    pending[1].wait()
    if j + 1 < _NCHUNK:
      nxt_x, nxt_q = bufs[(j + 1) % 2]
      off = base + (j + 1) * _CHUNK
      pending = (pltpu.async_copy(x_hbm.at[pl.ds(off, _CHUNK)], nxt_x, sem),
                 pltpu.async_copy(xsq_hbm.at[pl.ds(off, _CHUNK)], nxt_q, sem))
    pltpu.sync_copy(cur_x, s_sh.at[idx_v.at[j]], add=True)
    pltpu.sync_copy(cur_q, s2_sh.at[idx_v.at[j]], add=True)
    pltpu.sync_copy(ones_v, cnt_sh.at[idx_v.at[j]], add=True)

  plsc.subcore_barrier()
  pltpu.sync_copy(s_sh.at[zsl], s_out.at[core, zsl])
  pltpu.sync_copy(s2_sh.at[zsl], s2_out.at[core, zsl])
  pltpu.sync_copy(cnt_sh.at[zsl], cnt_out.at[core, zsl])


def _sc_gather_body(tab_hbm, ny2_hbm, g_out, g_v, idx_v, sg, ss):
  core = lax.axis_index("c")
  sid = lax.axis_index("s")
  wid = sid * _NC + core
  base = wid * _NCHUNK

  pltpu.sync_copy(ny2_hbm.at[pl.ds(base, _NCHUNK)], idx_v)
  gath = [None, None]
  stor = [None, None]
  for j in range(2):
    gath[j] = pltpu.async_copy(tab_hbm.at[idx_v.at[j]], g_v.at[j], sg)
  for j in range(_NCHUNK):
    b = j % 2
    gath[b].wait()
    stor[b] = pltpu.async_copy(g_v.at[b], g_out.at[base + j], ss)
    if j + 2 < _NCHUNK:
      stor[b].wait()
      gath[b] = pltpu.async_copy(tab_hbm.at[idx_v.at[j + 2]], g_v.at[b], sg)
  stor[0].wait()
  stor[1].wait()


def _tc_square_body(x_ref, o_ref):
  x = x_ref[...]
  o_ref[...] = x * x


def _tc_finalize_body(s_ref, s2_ref, c_ref, o_ref):
  s = s_ref[0] + s_ref[1]
  s2 = s2_ref[0] + s2_ref[1]
  cnt = c_ref[0, :, 0:1] + c_ref[1, :, 0:1]
  mean = s / cnt
  var = (s2 - cnt * mean * mean) / (cnt - 1.0)
  std = jnp.sqrt(jnp.maximum(var, 0.0))
  o_ref[:, 0:_D] = mean
  o_ref[:, _D:2 * _D] = std


def _tc_combine_body(x_ref, e_ref, g_ref, o_ref):
  g = g_ref[...]
  noise = g[:, 0:_D] + g[:, _D:2 * _D] * e_ref[...]
  o_ref[...] = (1.0 - _ALPHA) * x_ref[...] + _ALPHA * noise


def _segment_stats(x, xsq, y2, zeros, zeros16, ones):
  sds = jax.ShapeDtypeStruct((_NC, _CPAD, _D), jnp.float32)
  cds = jax.ShapeDtypeStruct((_NC, _CPAD, _CW), jnp.float32)
  k = pl.kernel(
      _sc_stats_body,
      out_type=(sds, sds, cds),
      mesh=_vmesh(),
      scratch_types=[
          pltpu.VMEM_SHARED((_CPAD, _D), jnp.float32),
          pltpu.VMEM_SHARED((_CPAD, _D), jnp.float32),
          pltpu.VMEM_SHARED((_CPAD, _CW), jnp.float32),
          pltpu.VMEM((_CHUNK, _D), jnp.float32),
          pltpu.VMEM((_CHUNK, _D), jnp.float32),
          pltpu.VMEM((_CHUNK, _D), jnp.float32),
          pltpu.VMEM((_CHUNK, _D), jnp.float32),
          pltpu.VMEM((_NCHUNK, _CHUNK), jnp.int32),
          pltpu.VMEM((_CHUNK, _CW), jnp.float32),
          pltpu.SemaphoreType.DMA,
      ],
  )
  return k(x, xsq, y2, zeros, zeros16, ones)


def _gather_rows(tab, ny2):
  k = pl.kernel(
      _sc_gather_body,
      out_type=jax.ShapeDtypeStruct((_N // _CHUNK, _CHUNK, 2 * _D),
                                    jnp.float32),
      mesh=_vmesh(),
      scratch_types=[
          pltpu.VMEM((2, _CHUNK, 2 * _D), jnp.float32),
          pltpu.VMEM((_NCHUNK, _CHUNK), jnp.int32),
          pltpu.SemaphoreType.DMA,
          pltpu.SemaphoreType.DMA,
      ],
  )
  return k(tab, ny2)


def _resample(y, key):
  k1, k2 = jax.random.split(key)
  perm = jax.random.permutation(k1, y.shape[0])
  ny = y[perm]

  k = k2
  for _ in range(1):
    k, sub = jax.random.split(k)
    rand = jax.random.randint(sub, y.shape, 0, _NUM_CLASSES).astype(y.dtype)
    ny = jnp.where(ny == y, rand, ny)

  def cond(state):
    n, _ = state
    return jnp.any(n == y)

  def body(state):
    n, kk = state
    kk, sub = jax.random.split(kk)
    rand = jax.random.randint(sub, y.shape, 0, _NUM_CLASSES).astype(y.dtype)
    n = jnp.where(n == y, rand, n)
    return (n, kk)

  ny, _ = jax.lax.while_loop(cond, body, (ny, k))
  return ny


def kernel(x, y):
  k_perm, k_noise = jax.random.split(jax.random.key(42))
  new_y = _resample(y, k_perm)

  blk = 1024
  xsq = pl.pallas_call(
      _tc_square_body,
      grid=(_N // blk,),
      in_specs=[pl.BlockSpec((blk, _D), lambda i: (i, 0))],
      out_specs=pl.BlockSpec((blk, _D), lambda i: (i, 0)),
      out_shape=jax.ShapeDtypeStruct((_N, _D), jnp.float32),
  )(x)

  zeros = jnp.zeros((_ZROWS, _D), jnp.float32)
  zeros16 = jnp.zeros((_ZROWS, _CW), jnp.float32)
  ones = jnp.ones((_CHUNK, _CW), jnp.float32)
  y2 = y.reshape(_N // _CHUNK, _CHUNK)
  s_p, s2_p, cnt_p = _segment_stats(x, xsq, y2, zeros, zeros16, ones)

  tab = pl.pallas_call(
      _tc_finalize_body,
      out_shape=jax.ShapeDtypeStruct((_CPAD, 2 * _D), jnp.float32),
  )(s_p, s2_p, cnt_p)

  ny2 = new_y.reshape(_N // _CHUNK, _CHUNK)
  gmgs = _gather_rows(tab, ny2).reshape(_N, 2 * _D)

  kd, _ = jax.lax.optimization_barrier((jax.random.key_data(k_noise), tab))
  eps = jax.random.normal(jax.random.wrap_key_data(kd), x.shape,
                          dtype=x.dtype)

  out = pl.pallas_call(
      _tc_combine_body,
      grid=(_N // blk,),
      in_specs=[
          pl.BlockSpec((blk, _D), lambda i: (i, 0)),
          pl.BlockSpec((blk, _D), lambda i: (i, 0)),
          pl.BlockSpec((blk, 2 * _D), lambda i: (i, 0)),
      ],
      out_specs=pl.BlockSpec((blk, _D), lambda i: (i, 0)),
      out_shape=jax.ShapeDtypeStruct((_N, _D), jnp.float32),
  )(x, eps, gmgs)

  return (out, new_y)


# speedup vs baseline: 1.4497x; 1.1566x over previous
import jax
import jax.numpy as jnp
from jax import lax
from jax.experimental import pallas as pl
from jax.experimental.pallas import tpu as pltpu
from jax.experimental.pallas import tpu_sc as plsc

_NUM_CLASSES = 1000
_ALPHA = 0.3

_N = 16384
_D = 128
_NC = 2
_NS = 16
_NW = _NC * _NS
_RPT = _N // _NW
_CHUNK = 128
_NCHUNK = _RPT // _CHUNK
_CPAD = 1024
_ZROWS = _CPAD // _NS
_CW = 16


def _vmesh():
  return plsc.VectorSubcoreMesh(core_axis_name="c", subcore_axis_name="s")


def _sc_stats_body(x_hbm, xsq_hbm, y2_hbm, zeros_hbm, zeros16_hbm, ones_hbm,
                   s_out, s2_out, cnt_out,
                   s_sh, s2_sh, cnt_sh, xa, xb, qa, qb, idx_v, ones_v, sem):
  core = lax.axis_index("c")
  sid = lax.axis_index("s")
  wid = sid * _NC + core
  base = wid * _RPT
  zsl = pl.ds(sid * _ZROWS, _ZROWS)

  pltpu.sync_copy(zeros_hbm, s_sh.at[zsl])
  pltpu.sync_copy(zeros_hbm, s2_sh.at[zsl])
  pltpu.sync_copy(zeros16_hbm, cnt_sh.at[zsl])
  pltpu.sync_copy(ones_hbm, ones_v)
  pltpu.sync_copy(y2_hbm.at[pl.ds(wid * _NCHUNK, _NCHUNK)], idx_v)
  plsc.subcore_barrier()

  bufs = ((xa, qa), (xb, qb))
  cp_x = pltpu.async_copy(x_hbm.at[pl.ds(base, _CHUNK)], xa, sem)
  cp_q = pltpu.async_copy(xsq_hbm.at[pl.ds(base, _CHUNK)], qa, sem)
  pending = (cp_x, cp_q)
  for j in range(_NCHUNK):
    cur_x, cur_q = bufs[j % 2]
    pending[0].wait()
    pending[1].wait()
    if j + 1 < _NCHUNK:
      nxt_x, nxt_q = bufs[(j + 1) % 2]
      off = base + (j + 1) * _CHUNK
      pending = (pltpu.async_copy(x_hbm.at[pl.ds(off, _CHUNK)], nxt_x, sem),
                 pltpu.async_copy(xsq_hbm.at[pl.ds(off, _CHUNK)], nxt_q, sem))
    pltpu.sync_copy(cur_x, s_sh.at[idx_v.at[j]], add=True)
    pltpu.sync_copy(cur_q, s2_sh.at[idx_v.at[j]], add=True)
    pltpu.sync_copy(ones_v, cnt_sh.at[idx_v.at[j]], add=True)

  plsc.subcore_barrier()
  pltpu.sync_copy(s_sh.at[zsl], s_out.at[core, zsl])
  pltpu.sync_copy(s2_sh.at[zsl], s2_out.at[core, zsl])
  pltpu.sync_copy(cnt_sh.at[zsl], cnt_out.at[core, zsl])


def _sc_gather_body(tab_hbm, ny2_hbm, g_out, g_v, idx_v, sg, ss):
  core = lax.axis_index("c")
  sid = lax.axis_index("s")
  wid = sid * _NC + core
  base = wid * _NCHUNK

  pltpu.sync_copy(ny2_hbm.at[pl.ds(base, _NCHUNK)], idx_v)
  gath = [None, None]
  stor = [None, None]
  for j in range(2):
    gath[j] = pltpu.async_copy(tab_hbm.at[idx_v.at[j]], g_v.at[j], sg)
  for j in range(_NCHUNK):
    b = j % 2
    gath[b].wait()
    stor[b] = pltpu.async_copy(g_v.at[b], g_out.at[base + j], ss)
    if j + 2 < _NCHUNK:
      stor[b].wait()
      gath[b] = pltpu.async_copy(tab_hbm.at[idx_v.at[j + 2]], g_v.at[b], sg)
  stor[0].wait()
  stor[1].wait()


def _tc_square_body(x_ref, o_ref):
  x = x_ref[...]
  o_ref[...] = x * x


def _tc_finalize_body(s_ref, s2_ref, c_ref, o_ref):
  s = s_ref[0] + s_ref[1]
  s2 = s2_ref[0] + s2_ref[1]
  cnt = c_ref[0, :, 0:1] + c_ref[1, :, 0:1]
  mean = s / cnt
  var = (s2 - cnt * mean * mean) / (cnt - 1.0)
  std = jnp.sqrt(jnp.maximum(var, 0.0))
  o_ref[:, 0:_D] = mean
  o_ref[:, _D:2 * _D] = std


def _tc_combine_body(x_ref, e_ref, g_ref, o_ref):
  g = g_ref[...]
  noise = g[:, 0:_D] + g[:, _D:2 * _D] * e_ref[...]
  o_ref[...] = (1.0 - _ALPHA) * x_ref[...] + _ALPHA * noise


def _segment_stats(x, xsq, y2, zeros, zeros16, ones):
  sds = jax.ShapeDtypeStruct((_NC, _CPAD, _D), jnp.float32)
  cds = jax.ShapeDtypeStruct((_NC, _CPAD, _CW), jnp.float32)
  k = pl.kernel(
      _sc_stats_body,
      out_type=(sds, sds, cds),
      mesh=_vmesh(),
      scratch_types=[
          pltpu.VMEM_SHARED((_CPAD, _D), jnp.float32),
          pltpu.VMEM_SHARED((_CPAD, _D), jnp.float32),
          pltpu.VMEM_SHARED((_CPAD, _CW), jnp.float32),
          pltpu.VMEM((_CHUNK, _D), jnp.float32),
          pltpu.VMEM((_CHUNK, _D), jnp.float32),
          pltpu.VMEM((_CHUNK, _D), jnp.float32),
          pltpu.VMEM((_CHUNK, _D), jnp.float32),
          pltpu.VMEM((_NCHUNK, _CHUNK), jnp.int32),
          pltpu.VMEM((_CHUNK, _CW), jnp.float32),
          pltpu.SemaphoreType.DMA,
      ],
  )
  return k(x, xsq, y2, zeros, zeros16, ones)


def _gather_rows(tab, ny2):
  k = pl.kernel(
      _sc_gather_body,
      out_type=jax.ShapeDtypeStruct((_N // _CHUNK, _CHUNK, 2 * _D),
                                    jnp.float32),
      mesh=_vmesh(),
      scratch_types=[
          pltpu.VMEM((2, _CHUNK, 2 * _D), jnp.float32),
          pltpu.VMEM((_NCHUNK, _CHUNK), jnp.int32),
          pltpu.SemaphoreType.DMA,
          pltpu.SemaphoreType.DMA,
      ],
  )
  return k(tab, ny2)


_KEYS = {}


def _key_consts():
  if not _KEYS:
    import numpy as np
    with jax.ensure_compile_time_eval():
      k_perm, k_noise = jax.random.split(jax.random.key(42))
      k1, k2 = jax.random.split(k_perm)
      kk = k1
      shuffle_subs = []
      for _ in range(2):
        kk, sub = jax.random.split(kk)
        shuffle_subs.append(np.asarray(jax.random.key_data(sub)))
      k_tail, sub1 = jax.random.split(k2)
      _KEYS.update(
          shuffle=shuffle_subs,
          round1=np.asarray(jax.random.key_data(sub1)),
          tail=np.asarray(jax.random.key_data(k_tail)),
          noise=np.asarray(jax.random.key_data(k_noise)),
      )
  return _KEYS


def _resample(y):
  keys = _key_consts()
  perm = jnp.arange(y.shape[0])
  for sub in keys["shuffle"]:
    sort_keys = jax.random.bits(jax.random.wrap_key_data(jnp.asarray(sub)),
                                (y.shape[0],), jnp.uint32)
    _, perm = lax.sort_key_val(sort_keys, perm, 0)
  ny = y[perm]

  rand = jax.random.randint(jax.random.wrap_key_data(jnp.asarray(keys["round1"])),
                            y.shape, 0, _NUM_CLASSES).astype(y.dtype)
  ny = jnp.where(ny == y, rand, ny)
  k = jax.random.wrap_key_data(jnp.asarray(keys["tail"]))

  def cond(state):
    n, _ = state
    return jnp.any(n == y)

  def body(state):
    n, kk = state
    kk, sub = jax.random.split(kk)
    rand = jax.random.randint(sub, y.shape, 0, _NUM_CLASSES).astype(y.dtype)
    n = jnp.where(n == y, rand, n)
    return (n, kk)

  ny, _ = jax.lax.while_loop(cond, body, (ny, k))
  return ny


def kernel(x, y):
  new_y = _resample(y)

  blk = 1024
  xsq = pl.pallas_call(
      _tc_square_body,
      grid=(_N // blk,),
      in_specs=[pl.BlockSpec((blk, _D), lambda i: (i, 0))],
      out_specs=pl.BlockSpec((blk, _D), lambda i: (i, 0)),
      out_shape=jax.ShapeDtypeStruct((_N, _D), jnp.float32),
  )(x)

  zeros = jnp.zeros((_ZROWS, _D), jnp.float32)
  zeros16 = jnp.zeros((_ZROWS, _CW), jnp.float32)
  ones = jnp.ones((_CHUNK, _CW), jnp.float32)
  y2 = y.reshape(_N // _CHUNK, _CHUNK)
  s_p, s2_p, cnt_p = _segment_stats(x, xsq, y2, zeros, zeros16, ones)

  tab = pl.pallas_call(
      _tc_finalize_body,
      out_shape=jax.ShapeDtypeStruct((_CPAD, 2 * _D), jnp.float32),
  )(s_p, s2_p, cnt_p)

  ny2 = new_y.reshape(_N // _CHUNK, _CHUNK)
  gmgs = _gather_rows(tab, ny2).reshape(_N, 2 * _D)

  kd, _ = jax.lax.optimization_barrier(
      (jnp.asarray(_key_consts()["noise"]), tab))
  eps = jax.random.normal(jax.random.wrap_key_data(kd), x.shape,
                          dtype=x.dtype)

  out = pl.pallas_call(
      _tc_combine_body,
      grid=(_N // blk,),
      in_specs=[
          pl.BlockSpec((blk, _D), lambda i: (i, 0)),
          pl.BlockSpec((blk, _D), lambda i: (i, 0)),
          pl.BlockSpec((blk, 2 * _D), lambda i: (i, 0)),
      ],
      out_specs=pl.BlockSpec((blk, _D), lambda i: (i, 0)),
      out_shape=jax.ShapeDtypeStruct((_N, _D), jnp.float32),
  )(x, eps, gmgs)

  return (out, new_y)
